# baseline (device time: 29326 ns/iter reference)
import jax
import jax.numpy as jnp
from jax import lax
from jax.experimental import pallas as pl
from jax.experimental.pallas import tpu as pltpu

N_DEV = 4


def kernel(x):
    m, n = x.shape

    def body(x_ref, out_ref, totals_ref, send_sems, recv_sems):
        my = lax.axis_index("i")

        barrier_sem = pltpu.get_barrier_semaphore()
        for off in range(1, N_DEV):
            pl.semaphore_signal(
                barrier_sem,
                inc=1,
                device_id=((my + off) % N_DEV,),
                device_id_type=pl.DeviceIdType.MESH,
            )
        pl.semaphore_wait(barrier_sem, N_DEV - 1)

        b = m // 8
        x3 = x_ref[:, :].reshape(b, 8, n)
        bidx = lax.broadcasted_iota(jnp.int32, (b, 8, n), 0)
        jidx = lax.broadcasted_iota(jnp.int32, (b, 8, n), 1)
        one = jnp.float32(1.0)
        for s in (1, 2, 4):
            t = pltpu.roll(x3, s, 1)
            tprev = pltpu.roll(t, 1, 0)
            x3 = x3 * jnp.where(
                jidx >= s, t, jnp.where(bidx == 0, one, tprev)
            )
        k = 1
        while k < b:
            t = pltpu.roll(x3, k, 0)
            x3 = x3 * jnp.where(bidx < k, one, t)
            k *= 2
        x = x3.reshape(m, n)
        out_ref[:, :] = x
        totals_ref[pl.ds(my, 1), :] = x[m - 1 : m, :]

        sends = []
        for k in range(N_DEV - 1):
            rdma = pltpu.make_async_remote_copy(
                src_ref=totals_ref.at[pl.ds(my, 1)],
                dst_ref=totals_ref.at[pl.ds(my, 1)],
                send_sem=send_sems.at[k],
                recv_sem=recv_sems.at[k],
                device_id=((my + k + 1) % N_DEV,),
                device_id_type=pl.DeviceIdType.MESH,
            )
            rdma.start()
            sends.append(rdma)

        for k in range(N_DEV - 1):
            src_row = (my - 1 - k) % N_DEV
            recv = pltpu.make_async_remote_copy(
                src_ref=totals_ref.at[pl.ds(src_row, 1)],
                dst_ref=totals_ref.at[pl.ds(src_row, 1)],
                send_sem=send_sems.at[k],
                recv_sem=recv_sems.at[k],
                device_id=(my,),
                device_id_type=pl.DeviceIdType.MESH,
            )
            recv.wait_recv()
        for rdma in sends:
            rdma.wait_send()

        totals = totals_ref[:, :]
        rid = lax.broadcasted_iota(jnp.int32, (N_DEV, n), 0)
        factors = jnp.where(rid < my, totals, jnp.ones_like(totals))
        prefix = factors[0] * factors[1] * factors[2] * factors[3]
        out_ref[:, :] = out_ref[:, :] * prefix[None, :]

    return pl.pallas_call(
        body,
        out_shape=jax.ShapeDtypeStruct((m, n), jnp.float32),
        in_specs=[pl.BlockSpec(memory_space=pltpu.VMEM)],
        out_specs=pl.BlockSpec(memory_space=pltpu.VMEM),
        scratch_shapes=[
            pltpu.VMEM((N_DEV, n), jnp.float32),
            pltpu.SemaphoreType.DMA((N_DEV - 1,)),
            pltpu.SemaphoreType.DMA((N_DEV - 1,)),
        ],
        compiler_params=pltpu.CompilerParams(collective_id=0),
    )(x)


# device time: 28715 ns/iter; 1.0213x vs baseline; 1.0213x over previous
import jax
import jax.numpy as jnp
from jax import lax
from jax.experimental import pallas as pl
from jax.experimental.pallas import tpu as pltpu

N_DEV = 4


def kernel(x):
    m, n = x.shape

    def body(x_ref, out_ref, totals_ref, send_sems, recv_sems):
        my = lax.axis_index("i")

        barrier_sem = pltpu.get_barrier_semaphore()
        for off in range(1, N_DEV):
            pl.semaphore_signal(
                barrier_sem,
                inc=1,
                device_id=((my + off) % N_DEV,),
                device_id_type=pl.DeviceIdType.MESH,
            )
        pl.semaphore_wait(barrier_sem, N_DEV - 1)

        r = x_ref[:, :]
        while r.shape[0] > 1:
            h = r.shape[0] // 2
            r = r[:h, :] * r[h:, :]
        totals_ref[pl.ds(my, 1), :] = r

        sends = []
        for k in range(N_DEV - 1):
            rdma = pltpu.make_async_remote_copy(
                src_ref=totals_ref.at[pl.ds(my, 1)],
                dst_ref=totals_ref.at[pl.ds(my, 1)],
                send_sem=send_sems.at[k],
                recv_sem=recv_sems.at[k],
                device_id=((my + k + 1) % N_DEV,),
                device_id_type=pl.DeviceIdType.MESH,
            )
            rdma.start()
            sends.append(rdma)

        b = m // 8
        x3 = x_ref[:, :].reshape(b, 8, n)
        bidx = lax.broadcasted_iota(jnp.int32, (b, 8, n), 0)
        jidx = lax.broadcasted_iota(jnp.int32, (b, 8, n), 1)
        one = jnp.float32(1.0)
        for s in (1, 2, 4):
            t = pltpu.roll(x3, s, 1)
            tprev = pltpu.roll(t, 1, 0)
            x3 = x3 * jnp.where(
                jidx >= s, t, jnp.where(bidx == 0, one, tprev)
            )
        k = 1
        while k < b:
            t = pltpu.roll(x3, k, 0)
            x3 = x3 * jnp.where(bidx < k, one, t)
            k *= 2
        x = x3.reshape(m, n)
        out_ref[:, :] = x

        for k in range(N_DEV - 1):
            src_row = (my - 1 - k) % N_DEV
            recv = pltpu.make_async_remote_copy(
                src_ref=totals_ref.at[pl.ds(src_row, 1)],
                dst_ref=totals_ref.at[pl.ds(src_row, 1)],
                send_sem=send_sems.at[k],
                recv_sem=recv_sems.at[k],
                device_id=(my,),
                device_id_type=pl.DeviceIdType.MESH,
            )
            recv.wait_recv()
        for rdma in sends:
            rdma.wait_send()

        totals = totals_ref[:, :]
        rid = lax.broadcasted_iota(jnp.int32, (N_DEV, n), 0)
        factors = jnp.where(rid < my, totals, jnp.ones_like(totals))
        prefix = factors[0] * factors[1] * factors[2] * factors[3]
        out_ref[:, :] = out_ref[:, :] * prefix[None, :]

    return pl.pallas_call(
        body,
        out_shape=jax.ShapeDtypeStruct((m, n), jnp.float32),
        in_specs=[pl.BlockSpec(memory_space=pltpu.VMEM)],
        out_specs=pl.BlockSpec(memory_space=pltpu.VMEM),
        scratch_shapes=[
            pltpu.VMEM((N_DEV, n), jnp.float32),
            pltpu.SemaphoreType.DMA((N_DEV - 1,)),
            pltpu.SemaphoreType.DMA((N_DEV - 1,)),
        ],
        compiler_params=pltpu.CompilerParams(collective_id=0),
    )(x)
